# baseline (speedup 1.0000x reference)
import functools

import jax
import jax.numpy as jnp
from jax import lax
from jax.experimental import pallas as pl
from jax.experimental.pallas import tpu as pltpu
from jax.experimental.pallas import tpu_sc as plsc

_B = 16384
_NC = 26
_NN = 13
_V = 100000
_D = 32
_H = 2 * _D
_NIDX = _B * _NC
_PK = 128 // _D

_NW = 32
_BPW = _NIDX // _NW
_CH = 416
_NCH = _BPW // _CH

_CAT_PROWS = _NIDX // _PK
_NUM_PROWS = _B * _NN // _PK
_RB_CAT = 2048
_RB_NUM = _B // _PK

_SQRT2 = 1.4142135623730951

_VP = 100096
_TLB = _VP // 17


def _tpose_body(x_ref, o_ref):
    x = x_ref[0]
    o_ref[:, 0:_D] = jnp.swapaxes(x, 0, 1).astype(jnp.bfloat16)


def _transpose_pack(Tn):
    return pl.pallas_call(
        _tpose_body,
        grid=(_NC, _VP // _TLB),
        in_specs=[pl.BlockSpec((1, _D, _TLB), lambda f, j: (f, 0, j))],
        out_specs=pl.BlockSpec((_TLB, 128), lambda f, j: (17 * f + j, 0)),
        out_shape=jax.ShapeDtypeStruct((_NC * _VP, 128), jnp.bfloat16),
        compiler_params=pltpu.CompilerParams(
            dimension_semantics=("parallel", "parallel")),
    )(Tn)


def _sc_gather(t128, idx):
    mesh = plsc.VectorSubcoreMesh(core_axis_name="core", subcore_axis_name="subcore")

    @functools.partial(
        pl.kernel,
        out_type=jax.ShapeDtypeStruct((_CAT_PROWS, 128), jnp.bfloat16),
        mesh=mesh,
        scratch_types=[pltpu.VMEM((_CH,), jnp.int32),
                       pltpu.VMEM((_CH,), jnp.int32),
                       pltpu.VMEM((_CH, 128), jnp.bfloat16),
                       pltpu.VMEM((_CH, 128), jnp.bfloat16),
                       pltpu.VMEM((_CH // _PK, 128), jnp.bfloat16),
                       pltpu.SemaphoreType.DMA,
                       pltpu.SemaphoreType.DMA],
        compiler_params=pltpu.CompilerParams(use_tc_tiling_on_sc=False),
    )
    def k(x_hbm, i_hbm, o_hbm, idx_v0, idx_v1, rows_v0, rows_v1, comp_v,
          sem0, sem1):
        wid = lax.axis_index("subcore") * 2 + lax.axis_index("core")
        base = wid * _BPW

        def start(c, idx_v, rows_v, sem):
            off = base + c * _CH
            pltpu.sync_copy(i_hbm.at[pl.ds(off, _CH)], idx_v)
            pltpu.async_copy(x_hbm.at[idx_v], rows_v, sem)

        def finish(c, idx_v, rows_v, sem):
            pltpu.make_async_copy(x_hbm.at[idx_v], rows_v, sem).wait()

            @pl.loop(0, _CH // _PK)
            def _(r):
                for a in range(_PK):
                    comp_v[r, pl.ds(a * _D, _D)] = (
                        rows_v[_PK * r + a, pl.ds(0, _D)])

            off = base + c * _CH
            pltpu.sync_copy(comp_v, o_hbm.at[pl.ds(off // _PK, _CH // _PK)])

        start(0, idx_v0, rows_v0, sem0)

        @pl.loop(0, _NCH // 2)
        def _(p):
            start(2 * p + 1, idx_v1, rows_v1, sem1)
            finish(2 * p, idx_v0, rows_v0, sem0)

            @pl.when(p < _NCH // 2 - 1)
            def _():
                start(2 * p + 2, idx_v0, rows_v0, sem0)

            finish(2 * p + 1, idx_v1, rows_v1, sem1)

    return k(t128, idx)


def _mlp_packed(x, w1, b1, w2, b2, g, be, m):
    h = jnp.dot(x, w1, preferred_element_type=jnp.float32) + b1
    h = 0.5 * h * (1.0 + lax.erf(h / _SQRT2))
    y = jnp.dot(h, w2, preferred_element_type=jnp.float32) + b2
    mu = jnp.dot(y, m, preferred_element_type=jnp.float32)
    c = y - mu
    var = jnp.dot(c * c, m, preferred_element_type=jnp.float32)
    return c * lax.rsqrt(var + 1e-5) * g + be


def _cat_body(x_ref, w1_ref, b1_ref, w2_ref, b2_ref, g_ref, be_ref, m_ref,
              o_ref):
    x = x_ref[...].astype(jnp.float32)
    o_ref[...] = _mlp_packed(x, w1_ref[...], b1_ref[...], w2_ref[...],
                             b2_ref[...], g_ref[...], be_ref[...], m_ref[...])


def _num_body(v_ref, e_ref, wn_ref, bn_ref, w1_ref, b1_ref, w2_ref, b2_ref,
              g_ref, be_ref, m_ref, o_ref):
    v = jnp.dot(v_ref[...], e_ref[...], preferred_element_type=jnp.float32)
    x = v * wn_ref[0] + bn_ref[0]
    o_ref[...] = _mlp_packed(x, w1_ref[...], b1_ref[...], w2_ref[...],
                             b2_ref[...], g_ref[...], be_ref[...], m_ref[...])


def _full(shape):
    return pl.BlockSpec(shape, lambda i: tuple(0 for _ in shape))


def kernel(cat_inputs, num_inputs, T, Wn, bn, W1, b1, W2, b2, gamma, beta):
    tp = _transpose_pack(T.transpose(0, 2, 1))
    idx = (cat_inputs.T.astype(jnp.int32)
           + (jnp.arange(_NC, dtype=jnp.int32) * _VP)[:, None]).reshape(_NIDX)

    cat_p = _sc_gather(tp, idx)

    eye4 = jnp.eye(_PK, dtype=jnp.float32)
    w1bd = jnp.kron(eye4, W1)
    w2bd = jnp.kron(eye4, W2)
    mavg = jnp.kron(eye4, jnp.full((_D, _D), 1.0 / _D, jnp.float32))
    b1t = jnp.tile(b1, _PK).reshape(1, _PK * _H)
    b2t = jnp.tile(b2, _PK).reshape(1, 128)
    gt = jnp.tile(gamma, _PK).reshape(1, 128)
    bet = jnp.tile(beta, _PK).reshape(1, 128)

    w_specs = [_full((128, _PK * _H)), _full((1, _PK * _H)),
               _full((_PK * _H, 128)), _full((1, 128)), _full((1, 128)),
               _full((1, 128)), _full((128, 128))]

    catm = pl.pallas_call(
        _cat_body,
        grid=(_CAT_PROWS // _RB_CAT,),
        in_specs=[pl.BlockSpec((_RB_CAT, 128), lambda i: (i, 0))] + w_specs,
        out_specs=pl.BlockSpec((_RB_CAT, 128), lambda i: (i, 0)),
        out_shape=jax.ShapeDtypeStruct((_CAT_PROWS, 128), jnp.float32),
    )(cat_p, w1bd, b1t, w2bd, b2t, gt, bet, mavg)

    num4 = num_inputs.T.reshape(_NUM_PROWS, _PK)
    e4 = jnp.kron(eye4, jnp.ones((1, _D), jnp.float32))
    wn_fm = jnp.tile(Wn, (1, _PK)).reshape(_NN, 1, 128)
    bn_fm = jnp.tile(bn, (1, _PK)).reshape(_NN, 1, 128)

    numm = pl.pallas_call(
        _num_body,
        grid=(_NN,),
        in_specs=[pl.BlockSpec((_RB_NUM, _PK), lambda i: (i, 0)),
                  _full((_PK, 128)),
                  pl.BlockSpec((1, 1, 128), lambda i: (i, 0, 0)),
                  pl.BlockSpec((1, 1, 128), lambda i: (i, 0, 0))] + w_specs,
        out_specs=pl.BlockSpec((_RB_NUM, 128), lambda i: (i, 0)),
        out_shape=jax.ShapeDtypeStruct((_NUM_PROWS, 128), jnp.float32),
    )(num4, e4, wn_fm, bn_fm, w1bd, b1t, w2bd, b2t, gt, bet, mavg)

    cat3 = catm.reshape(_NC, _B, _D).transpose(1, 0, 2)
    num3 = numm.reshape(_NN, _B, _D).transpose(1, 0, 2)
    return jnp.concatenate([cat3, num3], axis=1)


# speedup vs baseline: 2.7186x; 2.7186x over previous
import functools

import jax
import jax.numpy as jnp
from jax import lax
from jax.experimental import pallas as pl
from jax.experimental.pallas import tpu as pltpu
from jax.experimental.pallas import tpu_sc as plsc

_B = 16384
_NC = 26
_NN = 13
_V = 100000
_D = 32
_H = 2 * _D
_NIDX = _B * _NC
_PK = 128 // _D

_NW = 32
_BPW = _NIDX // _NW
_CH = 416
_NCH = _BPW // _CH

_CAT_PROWS = _NIDX // _PK
_NUM_PROWS = _B * _NN // _PK
_RB_CAT = 2048
_RB_NUM = _B // _PK

_SQRT2 = 1.4142135623730951

_VP = 100096
_TLB = _VP // 17


def _tpose_body(x_ref, o_ref):
    x = x_ref[0]
    o_ref[:, 0:_D] = jnp.swapaxes(x, 0, 1)


def _transpose_pack(Tn):
    return pl.pallas_call(
        _tpose_body,
        grid=(_NC, _VP // _TLB),
        in_specs=[pl.BlockSpec((1, _D, _TLB), lambda f, j: (f, 0, j))],
        out_specs=pl.BlockSpec((_TLB, 128), lambda f, j: (17 * f + j, 0)),
        out_shape=jax.ShapeDtypeStruct((_NC * _VP, 128), jnp.float32),
        compiler_params=pltpu.CompilerParams(
            dimension_semantics=("parallel", "parallel")),
    )(Tn)


def _sc_gather(t128, idx):
    mesh = plsc.VectorSubcoreMesh(core_axis_name="core", subcore_axis_name="subcore")

    @functools.partial(
        pl.kernel,
        out_type=jax.ShapeDtypeStruct((_CAT_PROWS, 128), jnp.float32),
        mesh=mesh,
        scratch_types=[pltpu.VMEM((_CH,), jnp.int32),
                       pltpu.VMEM((_CH,), jnp.int32),
                       pltpu.VMEM((_CH, 128), jnp.float32),
                       pltpu.VMEM((_CH, 128), jnp.float32),
                       pltpu.VMEM((_CH // _PK, 128), jnp.float32),
                       pltpu.SemaphoreType.DMA,
                       pltpu.SemaphoreType.DMA],
        compiler_params=pltpu.CompilerParams(use_tc_tiling_on_sc=False),
    )
    def k(x_hbm, i_hbm, o_hbm, idx_v0, idx_v1, rows_v0, rows_v1, comp_v,
          sem0, sem1):
        wid = lax.axis_index("subcore") * 2 + lax.axis_index("core")
        base = wid * _BPW

        def start(c, idx_v, rows_v, sem):
            off = base + c * _CH
            pltpu.sync_copy(i_hbm.at[pl.ds(off, _CH)], idx_v)
            pltpu.async_copy(x_hbm.at[idx_v], rows_v, sem)

        def finish(c, idx_v, rows_v, sem):
            pltpu.make_async_copy(x_hbm.at[idx_v], rows_v, sem).wait()

            @pl.loop(0, _CH // _PK)
            def _(r):
                for a in range(_PK):
                    for h in range(_D // 16):
                        comp_v[r, pl.ds(a * _D + h * 16, 16)] = (
                            rows_v[_PK * r + a, pl.ds(h * 16, 16)])

            off = base + c * _CH
            pltpu.sync_copy(comp_v, o_hbm.at[pl.ds(off // _PK, _CH // _PK)])

        start(0, idx_v0, rows_v0, sem0)

        @pl.loop(0, _NCH // 2)
        def _(p):
            start(2 * p + 1, idx_v1, rows_v1, sem1)
            finish(2 * p, idx_v0, rows_v0, sem0)

            @pl.when(p < _NCH // 2 - 1)
            def _():
                start(2 * p + 2, idx_v0, rows_v0, sem0)

            finish(2 * p + 1, idx_v1, rows_v1, sem1)

    return k(t128, idx)


def _mlp_packed(x, w1, b1, w2, b2, g, be, m):
    h = jnp.dot(x, w1, preferred_element_type=jnp.float32) + b1
    h = 0.5 * h * (1.0 + lax.erf(h / _SQRT2))
    y = jnp.dot(h, w2, preferred_element_type=jnp.float32) + b2
    mu = jnp.dot(y, m, preferred_element_type=jnp.float32)
    c = y - mu
    var = jnp.dot(c * c, m, preferred_element_type=jnp.float32)
    return c * lax.rsqrt(var + 1e-5) * g + be


def _cat_body(x_ref, w1_ref, b1_ref, w2_ref, b2_ref, g_ref, be_ref, m_ref,
              o_ref):
    o_ref[...] = _mlp_packed(x_ref[...], w1_ref[...], b1_ref[...], w2_ref[...],
                             b2_ref[...], g_ref[...], be_ref[...], m_ref[...])


def _num_body(v_ref, e_ref, wn_ref, bn_ref, w1_ref, b1_ref, w2_ref, b2_ref,
              g_ref, be_ref, m_ref, o_ref):
    v = jnp.dot(v_ref[...], e_ref[...], preferred_element_type=jnp.float32)
    x = v * wn_ref[0] + bn_ref[0]
    o_ref[...] = _mlp_packed(x, w1_ref[...], b1_ref[...], w2_ref[...],
                             b2_ref[...], g_ref[...], be_ref[...], m_ref[...])


def _full(shape):
    return pl.BlockSpec(shape, lambda i: tuple(0 for _ in shape))


def kernel(cat_inputs, num_inputs, T, Wn, bn, W1, b1, W2, b2, gamma, beta):
    tp = _transpose_pack(T.transpose(0, 2, 1))
    idx = (cat_inputs.T.astype(jnp.int32)
           + (jnp.arange(_NC, dtype=jnp.int32) * _VP)[:, None]).reshape(_NIDX)

    cat_p = _sc_gather(tp, idx)

    eye4 = jnp.eye(_PK, dtype=jnp.float32)
    w1bd = jnp.kron(eye4, W1)
    w2bd = jnp.kron(eye4, W2)
    mavg = jnp.kron(eye4, jnp.full((_D, _D), 1.0 / _D, jnp.float32))
    b1t = jnp.tile(b1, _PK).reshape(1, _PK * _H)
    b2t = jnp.tile(b2, _PK).reshape(1, 128)
    gt = jnp.tile(gamma, _PK).reshape(1, 128)
    bet = jnp.tile(beta, _PK).reshape(1, 128)

    w_specs = [_full((128, _PK * _H)), _full((1, _PK * _H)),
               _full((_PK * _H, 128)), _full((1, 128)), _full((1, 128)),
               _full((1, 128)), _full((128, 128))]

    catm = pl.pallas_call(
        _cat_body,
        grid=(_CAT_PROWS // _RB_CAT,),
        in_specs=[pl.BlockSpec((_RB_CAT, 128), lambda i: (i, 0))] + w_specs,
        out_specs=pl.BlockSpec((_RB_CAT, 128), lambda i: (i, 0)),
        out_shape=jax.ShapeDtypeStruct((_CAT_PROWS, 128), jnp.float32),
    )(cat_p, w1bd, b1t, w2bd, b2t, gt, bet, mavg)

    num4 = num_inputs.T.reshape(_NUM_PROWS, _PK)
    e4 = jnp.kron(eye4, jnp.ones((1, _D), jnp.float32))
    wn_fm = jnp.tile(Wn, (1, _PK)).reshape(_NN, 1, 128)
    bn_fm = jnp.tile(bn, (1, _PK)).reshape(_NN, 1, 128)

    numm = pl.pallas_call(
        _num_body,
        grid=(_NN,),
        in_specs=[pl.BlockSpec((_RB_NUM, _PK), lambda i: (i, 0)),
                  _full((_PK, 128)),
                  pl.BlockSpec((1, 1, 128), lambda i: (i, 0, 0)),
                  pl.BlockSpec((1, 1, 128), lambda i: (i, 0, 0))] + w_specs,
        out_specs=pl.BlockSpec((_RB_NUM, 128), lambda i: (i, 0)),
        out_shape=jax.ShapeDtypeStruct((_NUM_PROWS, 128), jnp.float32),
    )(num4, e4, wn_fm, bn_fm, w1bd, b1t, w2bd, b2t, gt, bet, mavg)

    cat3 = catm.reshape(_NC, _B, _D).transpose(1, 0, 2)
    num3 = numm.reshape(_NN, _B, _D).transpose(1, 0, 2)
    return jnp.concatenate([cat3, num3], axis=1)


# speedup vs baseline: 2.8362x; 1.0433x over previous
import functools

import jax
import jax.numpy as jnp
from jax import lax
from jax.experimental import pallas as pl
from jax.experimental.pallas import tpu as pltpu
from jax.experimental.pallas import tpu_sc as plsc

_B = 16384
_NC = 26
_NN = 13
_V = 100000
_D = 32
_H = 2 * _D
_NIDX = _B * _NC
_PK = 128 // _D

_NW = 32
_BPW = _NIDX // _NW
_CH = 416
_NCH = _BPW // _CH

_CAT_PROWS = _NIDX // _PK
_NUM_PROWS = _B * _NN // _PK
_RB_CAT = 2048
_RB_NUM = _B // _PK

_SQRT2 = 1.4142135623730951

_VP = 100096
_TLB = _VP // 17


def _tpose_body(x_ref, o_ref):
    x = x_ref[0]
    o_ref[:, 0:_D] = jnp.swapaxes(x, 0, 1)


def _transpose_pack(Tn):
    return pl.pallas_call(
        _tpose_body,
        grid=(_NC, _VP // _TLB),
        in_specs=[pl.BlockSpec((1, _D, _TLB), lambda f, j: (f, 0, j))],
        out_specs=pl.BlockSpec((_TLB, 128), lambda f, j: (17 * f + j, 0)),
        out_shape=jax.ShapeDtypeStruct((_NC * _VP, 128), jnp.float32),
        compiler_params=pltpu.CompilerParams(
            dimension_semantics=("parallel", "parallel")),
    )(Tn)


def _sc_gather(t128, idx):
    mesh = plsc.VectorSubcoreMesh(core_axis_name="core", subcore_axis_name="subcore")

    @functools.partial(
        pl.kernel,
        out_type=jax.ShapeDtypeStruct((_CAT_PROWS, 128), jnp.float32),
        mesh=mesh,
        scratch_types=[pltpu.VMEM((_CH,), jnp.int32),
                       pltpu.VMEM((_CH,), jnp.int32),
                       pltpu.VMEM((_CH, 128), jnp.float32),
                       pltpu.VMEM((_CH, 128), jnp.float32),
                       pltpu.VMEM((_CH // _PK, 128), jnp.float32),
                       pltpu.SemaphoreType.DMA,
                       pltpu.SemaphoreType.DMA],
        compiler_params=pltpu.CompilerParams(use_tc_tiling_on_sc=False),
    )
    def k(x_hbm, i_hbm, o_hbm, idx_v0, idx_v1, rows_v0, rows_v1, comp_v,
          sem0, sem1):
        wid = lax.axis_index("subcore") * 2 + lax.axis_index("core")
        base = wid * _BPW

        def start(c, idx_v, rows_v, sem):
            off = base + c * _CH
            pltpu.sync_copy(i_hbm.at[pl.ds(off, _CH)], idx_v)
            pltpu.async_copy(x_hbm.at[idx_v], rows_v, sem)

        def finish(c, idx_v, rows_v, sem):
            pltpu.make_async_copy(x_hbm.at[idx_v], rows_v, sem).wait()

            @pl.loop(0, _CH // _PK)
            def _(r):
                for a in range(_PK):
                    for h in range(_D // 16):
                        comp_v[r, pl.ds(a * _D + h * 16, 16)] = (
                            rows_v[_PK * r + a, pl.ds(h * 16, 16)])

            off = base + c * _CH
            pltpu.sync_copy(comp_v, o_hbm.at[pl.ds(off // _PK, _CH // _PK)])

        start(0, idx_v0, rows_v0, sem0)

        @pl.loop(0, _NCH // 2)
        def _(p):
            start(2 * p + 1, idx_v1, rows_v1, sem1)
            finish(2 * p, idx_v0, rows_v0, sem0)

            @pl.when(p < _NCH // 2 - 1)
            def _():
                start(2 * p + 2, idx_v0, rows_v0, sem0)

            finish(2 * p + 1, idx_v1, rows_v1, sem1)

    return k(t128, idx)


def _mlp_packed(x, w1, b1, w2, b2, g, be, m):
    h = jnp.dot(x, w1, preferred_element_type=jnp.float32) + b1
    h = 0.5 * h * (1.0 + lax.erf(h / _SQRT2))
    y = jnp.dot(h, w2, preferred_element_type=jnp.float32) + b2
    mu = jnp.dot(y, m, preferred_element_type=jnp.float32)
    c = y - mu
    var = jnp.dot(c * c, m, preferred_element_type=jnp.float32)
    return c * lax.rsqrt(var + 1e-5) * g + be


def _cat_body(x_ref, w1_ref, b1_ref, w2_ref, b2_ref, g_ref, be_ref, m_ref,
              o_ref):
    o_ref[...] = _mlp_packed(x_ref[...], w1_ref[...], b1_ref[...], w2_ref[...],
                             b2_ref[...], g_ref[...], be_ref[...], m_ref[...])


def _num_body(v_ref, wn_ref, bn_ref, w1t_ref, b1c_ref, w2t_ref, b2c_ref,
              gc_ref, bec_ref, o_ref):
    x = v_ref[0]
    tok = wn_ref[0] * x + bn_ref[0]
    h = jnp.dot(w1t_ref[...], tok,
                preferred_element_type=jnp.float32) + b1c_ref[...]
    h = 0.5 * h * (1.0 + lax.erf(h / _SQRT2))
    y = jnp.dot(w2t_ref[...], h,
                preferred_element_type=jnp.float32) + b2c_ref[...]
    mu = jnp.mean(y, axis=0, keepdims=True)
    c = y - mu
    var = jnp.mean(c * c, axis=0, keepdims=True)
    o_ref[0] = c * lax.rsqrt(var + 1e-5) * gc_ref[...] + bec_ref[...]


def _full(shape):
    return pl.BlockSpec(shape, lambda i: tuple(0 for _ in shape))


def kernel(cat_inputs, num_inputs, T, Wn, bn, W1, b1, W2, b2, gamma, beta):
    tp = _transpose_pack(T.transpose(0, 2, 1))
    idx = (cat_inputs.T.astype(jnp.int32)
           + (jnp.arange(_NC, dtype=jnp.int32) * _VP)[:, None]).reshape(_NIDX)

    cat_p = _sc_gather(tp, idx)

    eye4 = jnp.eye(_PK, dtype=jnp.float32)
    w1bd = jnp.kron(eye4, W1)
    w2bd = jnp.kron(eye4, W2)
    mavg = jnp.kron(eye4, jnp.full((_D, _D), 1.0 / _D, jnp.float32))
    b1t = jnp.tile(b1, _PK).reshape(1, _PK * _H)
    b2t = jnp.tile(b2, _PK).reshape(1, 128)
    gt = jnp.tile(gamma, _PK).reshape(1, 128)
    bet = jnp.tile(beta, _PK).reshape(1, 128)

    w_specs = [_full((128, _PK * _H)), _full((1, _PK * _H)),
               _full((_PK * _H, 128)), _full((1, 128)), _full((1, 128)),
               _full((1, 128)), _full((128, 128))]

    catm = pl.pallas_call(
        _cat_body,
        grid=(_CAT_PROWS // _RB_CAT,),
        in_specs=[pl.BlockSpec((_RB_CAT, 128), lambda i: (i, 0))] + w_specs,
        out_specs=pl.BlockSpec((_RB_CAT, 128), lambda i: (i, 0)),
        out_shape=jax.ShapeDtypeStruct((_CAT_PROWS, 128), jnp.float32),
    )(cat_p, w1bd, b1t, w2bd, b2t, gt, bet, mavg)

    numT3 = num_inputs.T.reshape(_NN, 1, _B)
    wn_c = Wn.reshape(_NN, _D, 1)
    bn_c = bn.reshape(_NN, _D, 1)
    w1t = W1.T
    w2t = W2.T
    b1c = b1.reshape(_H, 1)
    b2c = b2.reshape(_D, 1)
    gc = gamma.reshape(_D, 1)
    bec = beta.reshape(_D, 1)

    numm = pl.pallas_call(
        _num_body,
        grid=(_NN,),
        in_specs=[pl.BlockSpec((1, 1, _B), lambda i: (i, 0, 0)),
                  pl.BlockSpec((1, _D, 1), lambda i: (i, 0, 0)),
                  pl.BlockSpec((1, _D, 1), lambda i: (i, 0, 0)),
                  _full((_H, _D)), _full((_H, 1)), _full((_D, _H)),
                  _full((_D, 1)), _full((_D, 1)), _full((_D, 1))],
        out_specs=pl.BlockSpec((1, _D, _B), lambda i: (i, 0, 0)),
        out_shape=jax.ShapeDtypeStruct((_NN, _D, _B), jnp.float32),
    )(numT3, wn_c, bn_c, w1t, b1c, w2t, b2c, gc, bec)

    cat3 = catm.reshape(_NC, _B, _D).transpose(1, 0, 2)
    num3 = numm.transpose(2, 0, 1)
    return jnp.concatenate([cat3, num3], axis=1)
